# baseline (speedup 1.0000x reference)
import functools

import jax
import jax.numpy as jnp
from jax import lax
from jax.experimental import pallas as pl
from jax.experimental.pallas import tpu as pltpu
from jax.experimental.pallas import tpu_sc as plsc


def kernel(input_tensor):
    B, C, H, W = input_tensor.shape
    D = H * W
    R = B * (C // 2)
    x = input_tensor.reshape(R, 2, D)

    NC, NS = 2, 16
    NW = NC * NS
    rows_per_w = R // NW

    mesh = plsc.VectorSubcoreMesh(core_axis_name="c", subcore_axis_name="s")

    @functools.partial(
        pl.kernel,
        mesh=mesh,
        out_type=jax.ShapeDtypeStruct((R, 1, D), jnp.float32),
    )
    def sel(in_hbm, out_hbm):
        wid = lax.axis_index("s") * NC + lax.axis_index("c")
        base = wid * rows_per_w
        pltpu.sync_copy(
            in_hbm.at[pl.ds(base, rows_per_w), pl.ds(0, 1), :],
            out_hbm.at[pl.ds(base, rows_per_w)],
        )

    y = sel(x)
    return y.reshape(B, C // 2, H, W)


# speedup vs baseline: 2.9799x; 2.9799x over previous
import functools

import jax
import jax.numpy as jnp
from jax import lax
from jax.experimental import pallas as pl
from jax.experimental.pallas import tpu as pltpu
from jax.experimental.pallas import tpu_sc as plsc


def kernel(input_tensor):
    B, C, H, W = input_tensor.shape
    D = H * W
    R = B * (C // 2)
    x = input_tensor.reshape(R, 2, D)

    NC, NS = 2, 16
    NW = NC * NS
    rows_per_w = R // NW

    NB = 2
    CH = 64
    nch = rows_per_w // CH

    mesh = plsc.VectorSubcoreMesh(core_axis_name="c", subcore_axis_name="s")

    @functools.partial(
        pl.kernel,
        mesh=mesh,
        out_type=jax.ShapeDtypeStruct((R, 1, D), jnp.float32),
        scratch_types=[
            pltpu.VMEM((NB, CH, 1, D), jnp.float32),
            pltpu.SemaphoreType.DMA,
            pltpu.SemaphoreType.DMA,
            pltpu.SemaphoreType.DMA,
            pltpu.SemaphoreType.DMA,
        ],
    )
    def sel(in_hbm, out_hbm, buf, g0, g1, s0, s1):
        gsem = [g0, g1]
        ssem = [s0, s1]
        wid = lax.axis_index("s") * NC + lax.axis_index("c")
        base = wid * rows_per_w
        gath = [None] * NB
        scat = [None] * NB
        for k in range(nch):
            b = k % NB
            if scat[b] is not None:
                scat[b].wait()
            gath[b] = pltpu.async_copy(
                in_hbm.at[pl.ds(base + k * CH, CH), pl.ds(0, 1), :],
                buf.at[b],
                gsem[b],
            )
            if k >= 1:
                pb = (k - 1) % NB
                gath[pb].wait()
                scat[pb] = pltpu.async_copy(
                    buf.at[pb],
                    out_hbm.at[pl.ds(base + (k - 1) * CH, CH)],
                    ssem[pb],
                )
        lb = (nch - 1) % NB
        gath[lb].wait()
        scat[lb] = pltpu.async_copy(
            buf.at[lb],
            out_hbm.at[pl.ds(base + (nch - 1) * CH, CH)],
            ssem[lb],
        )
        for b in range(NB):
            if scat[b] is not None:
                scat[b].wait()

    y = sel(x)
    return y.reshape(B, C // 2, H, W)
